# Initial kernel scaffold; baseline (speedup 1.0000x reference)
import jax
import jax.numpy as jnp
from jax.experimental import pallas as pl


def kernel(inputs, table):
    raise NotImplementedError("write your pallas kernel here")


# speedup vs baseline: 1.4776x; 1.4776x over previous
import functools

import jax
import jax.numpy as jnp
from jax import lax
from jax.experimental import pallas as pl
from jax.experimental.pallas import tpu as pltpu
from jax.experimental.pallas import tpu_sc as plsc

VOCAB = 1000000
EMBED_DIM = 32
BATCH = 4096
MAX_LEN = 200

_INFO = plsc.get_sparse_core_info()
_NC = _INFO.num_cores
_NS = _INFO.num_subcores
_NW = _NC * _NS

_N = BATCH * MAX_LEN
_PER_W = _N // _NW
_IDX_MINOR = 128
_ROWS_PER_STEP = 1024
_STREAMS_PER_STEP = _ROWS_PER_STEP // _IDX_MINOR
_STEPS = _PER_W // _ROWS_PER_STEP
_IDX_ROWS = _PER_W // _IDX_MINOR


def _make_kernel():
    mesh = plsc.VectorSubcoreMesh(core_axis_name="c", subcore_axis_name="s")

    @functools.partial(
        pl.kernel,
        mesh=mesh,
        compiler_params=pltpu.CompilerParams(use_tc_tiling_on_sc=False),
        out_type=jax.ShapeDtypeStruct((_N, EMBED_DIM), jnp.float32),
        scratch_types=[
            pltpu.VMEM((_IDX_ROWS, _IDX_MINOR), jnp.int32),
            pltpu.VMEM((_ROWS_PER_STEP, EMBED_DIM), jnp.float32),
            pltpu.SemaphoreType.DMA,
        ],
    )
    def emb_kernel(idx_hbm, table_hbm, out_hbm, idx_v, rows_v, sem):
        wid = lax.axis_index("s") * _NC + lax.axis_index("c")
        pltpu.sync_copy(idx_hbm.at[wid], idx_v)
        out_base = wid * _PER_W

        def step(g, _):
            copies = []
            for j in range(_STREAMS_PER_STEP):
                copies.append(
                    pltpu.async_copy(
                        table_hbm.at[idx_v.at[g * _STREAMS_PER_STEP + j]],
                        rows_v.at[pl.ds(j * _IDX_MINOR, _IDX_MINOR)],
                        sem,
                    )
                )
            for c in copies:
                c.wait()
            pltpu.sync_copy(
                rows_v,
                out_hbm.at[pl.ds(out_base + g * _ROWS_PER_STEP, _ROWS_PER_STEP)],
            )
            return 0

        lax.fori_loop(0, _STEPS, step, 0)

    return emb_kernel


_EMB_KERNEL = _make_kernel()


@jax.jit
def kernel(inputs, table):
    idx = inputs.astype(jnp.int32).reshape(_NW, _IDX_ROWS, _IDX_MINOR)
    out = _EMB_KERNEL(idx, table)
    return out.reshape(BATCH, MAX_LEN, EMBED_DIM)


# speedup vs baseline: 1.4914x; 1.0093x over previous
import functools

import jax
import jax.numpy as jnp
from jax import lax
from jax.experimental import pallas as pl
from jax.experimental.pallas import tpu as pltpu
from jax.experimental.pallas import tpu_sc as plsc

VOCAB = 1000000
EMBED_DIM = 32
BATCH = 4096
MAX_LEN = 200

_INFO = plsc.get_sparse_core_info()
_NC = _INFO.num_cores
_NS = _INFO.num_subcores
_NW = _NC * _NS

_N = BATCH * MAX_LEN
_PER_W = _N // _NW
_IDX_MINOR = 128
_ROWS = 1280
_NSTR = _ROWS // _IDX_MINOR
_STEPS = _PER_W // _ROWS
_PAIRS = _STEPS // 2
_IDX_ROWS = _PER_W // _IDX_MINOR


def _make_kernel():
    mesh = plsc.VectorSubcoreMesh(core_axis_name="c", subcore_axis_name="s")

    @functools.partial(
        pl.kernel,
        mesh=mesh,
        compiler_params=pltpu.CompilerParams(use_tc_tiling_on_sc=False),
        out_type=jax.ShapeDtypeStruct((_N, EMBED_DIM), jnp.float32),
        scratch_types=[
            pltpu.VMEM((_IDX_ROWS, _IDX_MINOR), jnp.int32),
            pltpu.VMEM((_ROWS, EMBED_DIM), jnp.float32),
            pltpu.VMEM((_ROWS, EMBED_DIM), jnp.float32),
            pltpu.SemaphoreType.DMA,
            pltpu.SemaphoreType.DMA,
            pltpu.SemaphoreType.DMA,
            pltpu.SemaphoreType.DMA,
        ],
    )
    def emb_kernel(idx_hbm, table_hbm, out_hbm, idx_v, rows0, rows1,
                   sg0, sg1, ss0, ss1):
        wid = lax.axis_index("s") * _NC + lax.axis_index("c")
        pltpu.sync_copy(idx_hbm.at[wid], idx_v)
        out_base = wid * _PER_W

        def fire(c, buf, sem):
            for j in range(_NSTR):
                pltpu.async_copy(
                    table_hbm.at[idx_v.at[c * _NSTR + j]],
                    buf.at[pl.ds(j * _IDX_MINOR, _IDX_MINOR)],
                    sem,
                )

        def drain_gather(buf, sem):
            pltpu.make_async_copy(out_hbm.at[pl.ds(0, _ROWS)], buf, sem).wait()

        def store_start(buf, c, sem):
            pltpu.async_copy(buf, out_hbm.at[pl.ds(out_base + c * _ROWS, _ROWS)], sem)

        def store_wait(buf, sem):
            pltpu.make_async_copy(buf, out_hbm.at[pl.ds(0, _ROWS)], sem).wait()

        fire(0, rows0, sg0)
        fire(1, rows1, sg1)

        def pair(p, _):
            c0 = 2 * p
            drain_gather(rows0, sg0)
            store_start(rows0, c0, ss0)
            drain_gather(rows1, sg1)
            store_start(rows1, c0 + 1, ss1)
            store_wait(rows0, ss0)
            fire(c0 + 2, rows0, sg0)
            store_wait(rows1, ss1)
            fire(c0 + 3, rows1, sg1)
            return 0

        lax.fori_loop(0, _PAIRS - 1, pair, 0)

        c0 = _STEPS - 2
        drain_gather(rows0, sg0)
        store_start(rows0, c0, ss0)
        drain_gather(rows1, sg1)
        store_start(rows1, c0 + 1, ss1)
        store_wait(rows0, ss0)
        store_wait(rows1, ss1)

    return emb_kernel


_EMB_KERNEL = _make_kernel()


@jax.jit
def kernel(inputs, table):
    idx = inputs.astype(jnp.int32).reshape(_NW, _IDX_ROWS, _IDX_MINOR)
    out = _EMB_KERNEL(idx, table)
    return out.reshape(BATCH, MAX_LEN, EMBED_DIM)
